# baseline (speedup 1.0000x reference)
import functools

import numpy as np
import jax
import jax.numpy as jnp
from jax import lax
from jax.experimental import pallas as pl
from jax.experimental.pallas import tpu as pltpu
from jax.experimental.pallas import tpu_sc as plsc

NE = 10000
NN = 1000
EV = 160000
EP = 163840
D = 128
ROWB = 1024
NB_T = EP // ROWB
ZR = 10240
DUMMY = 10200
CUTOFF = 10.0
DIST_FEAT = 16
LOG2 = float(np.log(2.0))

def _ssp(x):
    return jnp.maximum(x, 0.0) + jnp.log1p(jnp.exp(-jnp.abs(x))) - LOG2


def _pad2(w, r, c):
    return jnp.zeros((r, c), jnp.float32).at[: w.shape[0], : w.shape[1]].set(w)


def _pad1(b, c):
    return jnp.zeros((1, c), jnp.float32).at[0, : b.shape[0]].set(b)


def _expand_body(s_ref, r_ref, o_ref):
    li = lax.broadcasted_iota(jnp.int32, (1, D), 1)
    q = (li % DIST_FEAT).astype(jnp.float32) / (DIST_FEAT - 1.0)
    valid = li < 7 * DIST_FEAT
    mus = jnp.where(valid, CUTOFF * q * q, 0.0)
    sig = (1.0 + CUTOFF * q) / 7.0
    isig = jnp.where(valid, 1.0 / (sig * sig), 0.0)
    cols = []
    d_sq = None
    for i in range(3):
        di = r_ref[:, i : i + 1] - s_ref[:, i : i + 1]
        d_sq = di * di if d_sq is None else d_sq + di * di
        cols.append(jnp.maximum(di, 0.0))
        cols.append(jnp.maximum(-di, 0.0))
    cols.append(jnp.sqrt(d_sq))
    x = jnp.concatenate(
        [jnp.broadcast_to(c, (ROWB, DIST_FEAT)) for c in cols]
        + [jnp.zeros((ROWB, DIST_FEAT), jnp.float32)],
        axis=1,
    )
    env = x * x * jnp.exp(-x)
    dm = x - mus
    o_ref[...] = (env * jnp.exp(-(dm * dm) * isig)).astype(jnp.bfloat16)


def _expand_call(posg):
    return pl.pallas_call(
        _expand_body,
        grid=(3 * NB_T,),
        in_specs=[
            pl.BlockSpec((ROWB, 16), lambda i: (3 * NB_T + i, 0)),
            pl.BlockSpec((ROWB, 16), lambda i: (i, 0)),
        ],
        out_specs=pl.BlockSpec((ROWB, D), lambda i: (i, 0)),
        out_shape=jax.ShapeDtypeStruct((3 * EP, D), jnp.bfloat16),
    )(posg, posg)


def _mlp2_body(x_ref, w1_ref, b1_ref, w2_ref, o_ref):
    a = _ssp(jnp.dot(x_ref[...], w1_ref[...],
                     preferred_element_type=jnp.float32) + b1_ref[...])
    o_ref[...] = jnp.dot(a, w2_ref[...],
                         preferred_element_type=jnp.float32).astype(o_ref.dtype)


def _mlp2_call(x, w1, b1, w2, out_dtype=jnp.float32):
    n = x.shape[0]
    blk = 1000
    return pl.pallas_call(
        _mlp2_body,
        grid=(n // blk,),
        in_specs=[
            pl.BlockSpec((blk, D), lambda i: (i, 0)),
            pl.BlockSpec((D, D), lambda i: (0, 0)),
            pl.BlockSpec((1, D), lambda i: (0, 0)),
            pl.BlockSpec((D, D), lambda i: (0, 0)),
        ],
        out_specs=pl.BlockSpec((blk, D), lambda i: (i, 0)),
        out_shape=jax.ShapeDtypeStruct((n, D), out_dtype),
    )(x, w1, b1, w2)


def _we_body(e_ref, g_ref, w1, b1, w2, b2, w3, o_ref):
    a = _ssp(jnp.dot(e_ref[...], w1[...], preferred_element_type=jnp.float32)
             + b1[...]).astype(jnp.bfloat16)
    a = _ssp(jnp.dot(a, w2[...], preferred_element_type=jnp.float32)
             + b2[...]).astype(jnp.bfloat16)
    o_ref[...] = (jnp.dot(a, w3[...], preferred_element_type=jnp.float32)
                  * g_ref[...].astype(jnp.float32))


def _we_call(edata, t_idx, gath, w1, b1, w2, b2, w3):
    return pl.pallas_call(
        _we_body,
        grid=(NB_T,),
        in_specs=[
            pl.BlockSpec((ROWB, D), lambda i: (t_idx * NB_T + i, 0)),
            pl.BlockSpec((ROWB, D), lambda i: (i, 0)),
            pl.BlockSpec((D, D), lambda i: (0, 0)),
            pl.BlockSpec((1, D), lambda i: (0, 0)),
            pl.BlockSpec((D, D), lambda i: (0, 0)),
            pl.BlockSpec((1, D), lambda i: (0, 0)),
            pl.BlockSpec((D, D), lambda i: (0, 0)),
        ],
        out_specs=pl.BlockSpec((ROWB, D), lambda i: (i, 0)),
        out_shape=jax.ShapeDtypeStruct((EP, D), jnp.float32),
    )(edata, gath, w1, b1, w2, b2, w3)


def _upd_body(elec_ref, z0, z1, z2, w1, b1, w2, o_ref):
    acc = elec_ref[...]
    for t, zr in enumerate((z0, z1, z2)):
        zt = zr[0] + zr[1]
        a = _ssp(jnp.dot(zt, w1[t], preferred_element_type=jnp.float32) + b1[t])
        acc = acc + jnp.dot(a, w2[t], preferred_element_type=jnp.float32)
    o_ref[...] = acc


def _upd_call(elec, zs, w1, b1, w2):
    blk = 1000
    zspec = pl.BlockSpec((2, blk, D), lambda i: (0, i, 0))
    return pl.pallas_call(
        _upd_body,
        grid=(NE // blk,),
        in_specs=[
            pl.BlockSpec((blk, D), lambda i: (i, 0)),
            zspec, zspec, zspec,
            pl.BlockSpec((3, D, D), lambda i: (0, 0, 0)),
            pl.BlockSpec((3, 1, D), lambda i: (0, 0, 0)),
            pl.BlockSpec((3, D, D), lambda i: (0, 0, 0)),
        ],
        out_specs=pl.BlockSpec((blk, D), lambda i: (i, 0)),
        out_shape=jax.ShapeDtypeStruct((NE, D), jnp.float32),
    )(elec, zs[0], zs[1], zs[2], w1, b1, w2)


def _mesh():
    return plsc.VectorSubcoreMesh(core_axis_name="c", subcore_axis_name="s")


NBUF = 4


@functools.partial(functools.lru_cache)
def _make_gather(dcols, b_total, dtype=jnp.float32):
    per_w = b_total // 32
    n_chunks = per_w // 128
    n_groups = n_chunks // NBUF

    @functools.partial(
        pl.kernel,
        out_type=jax.ShapeDtypeStruct((b_total, dcols), dtype),
        mesh=_mesh(),
        compiler_params=pltpu.CompilerParams(use_tc_tiling_on_sc=False),
        scratch_types=[
            pltpu.VMEM((n_chunks, 128), jnp.int32),
            pltpu.VMEM((NBUF, 128, dcols), dtype),
        ] + [pltpu.SemaphoreType.DMA] * NBUF,
    )
    def gk(table_hbm, idx_hbm, out_hbm, idx_all, rows, *sems):
        c = lax.axis_index("c")
        s = lax.axis_index("s")
        wid = s * 2 + c
        base = wid * per_w
        pltpu.sync_copy(idx_hbm.at[pl.ds(wid * n_chunks, n_chunks)], idx_all)

        def gstart(b, g):
            pltpu.async_copy(table_hbm.at[idx_all.at[g]], rows.at[b], sems[b])

        def gwait(b):
            pltpu.make_async_copy(
                out_hbm.at[pl.ds(base, 128)], rows.at[b], sems[b]).wait()

        for b in range(NBUF):
            gstart(b, b)

        def grp(k, carry):
            for b in range(NBUF):
                gwait(b)
                g = k * NBUF + b
                pltpu.async_copy(
                    rows.at[b], out_hbm.at[pl.ds(base + g * 128, 128)], sems[b])
            for b in range(NBUF):
                gwait(b)

            @pl.when(k + 1 < n_groups)
            def _():
                for b in range(NBUF):
                    gstart(b, (k + 1) * NBUF + b)
            return carry

        lax.fori_loop(0, n_groups, grp, 0)

    return gk


@functools.partial(functools.lru_cache)
def _make_scatter():
    per_w = EP // 32
    n_chunks = per_w // 128
    snbuf = 2
    n_groups = n_chunks // snbuf
    tile_rows = ZR // 16

    @functools.partial(
        pl.kernel,
        out_type=jax.ShapeDtypeStruct((2, ZR, D), jnp.float32),
        mesh=_mesh(),
        scratch_types=[
            pltpu.VMEM((n_chunks, 128), jnp.int32),
            pltpu.VMEM((snbuf, 128, D), jnp.float32),
            pltpu.VMEM_SHARED((ZR, D), jnp.float32),
        ] + [pltpu.SemaphoreType.DMA] * snbuf,
    )
    def sk(weh, idx_hbm, out_hbm, idx_all, rows, acc, *sems):
        c = lax.axis_index("c")
        s = lax.axis_index("s")
        wid = c * 16 + s
        base = c * (EP // 2) + s * per_w
        pltpu.sync_copy(idx_hbm.at[pl.ds(wid * n_chunks, n_chunks)], idx_all)

        def zb(i, carry):
            for k in range(8):
                rows[0, i, pl.ds(k * 16, 16)] = jnp.zeros((16,), jnp.float32)
            return carry

        lax.fori_loop(0, 128, zb, 0)
        tile_r0 = s * tile_rows
        for bb in range(tile_rows // 128):
            pltpu.sync_copy(rows.at[0], acc.at[pl.ds(tile_r0 + bb * 128, 128)])
        plsc.subcore_barrier()

        def lstart(b, g):
            pltpu.async_copy(
                weh.at[pl.ds(base + g * 128, 128)], rows.at[b], sems[b])

        def swait(b):
            pltpu.make_async_copy(
                weh.at[pl.ds(base, 128)], rows.at[b], sems[b]).wait()

        for b in range(snbuf):
            lstart(b, b)

        def grp(k, carry):
            for b in range(snbuf):
                swait(b)
                g = k * snbuf + b
                pltpu.async_copy(rows.at[b], acc.at[idx_all.at[g]], sems[b],
                                 add=True)
            for b in range(snbuf):
                swait(b)

            @pl.when(k + 1 < n_groups)
            def _():
                for b in range(snbuf):
                    lstart(b, (k + 1) * snbuf + b)
            return carry

        lax.fori_loop(0, n_groups, grp, 0)
        plsc.subcore_barrier()

        for bb in range(tile_rows // 128):
            r0 = tile_r0 + bb * 128
            pltpu.sync_copy(acc.at[pl.ds(r0, 128)], rows.at[0])
            pltpu.sync_copy(rows.at[0], out_hbm.at[c, pl.ds(r0, 128)])

    return sk


def _pad_idx(a, n, fill):
    a = a.astype(jnp.int32)
    return jnp.concatenate([a, jnp.full((n - a.shape[0],), fill, jnp.int32)])


def _chunked(idx):
    return idx.reshape(idx.shape[0] // 128, 128)


def kernel(params, rs, coords, senders_ne, receivers_ne, senders_same,
           receivers_same, senders_anti, receivers_anti):
    senders = {"same": senders_same, "anti": senders_anti, "ne": senders_ne}
    receivers = {"same": receivers_same, "anti": receivers_anti,
                 "ne": receivers_ne}

    tpos = jnp.concatenate(
        [_pad2(rs, NE, 16), _pad2(coords, NN, 16)], axis=0)
    rpos_idx = jnp.concatenate(
        [_pad_idx(receivers[l], EP, receivers[l][0]) for l in
         ("same", "anti", "ne")])
    spos_idx = jnp.concatenate([
        _pad_idx(senders["same"], EP, senders["same"][0]),
        _pad_idx(senders["anti"], EP, senders["anti"][0]),
        _pad_idx(senders["ne"], EP, senders["ne"][0]) + NE,
    ])
    pos_idx = _chunked(jnp.concatenate([rpos_idx, spos_idx]))
    sg_idx = {l: _chunked(_pad_idx(senders[l], EP, 0)) for l in ("same", "anti")}
    yg_idx = _chunked(_pad_idx(senders["ne"], EP, 0))
    scat_idx = {l: _chunked(_pad_idx(receivers[l], EP, DUMMY))
                for l in ("same", "anti", "ne")}

    wl = []
    for lp in params["layers"]:
        ent = {}
        for nm in ("w1", "b1", "w2", "b2", "w3"):
            ent[nm] = []
        for lbl in ("same", "anti", "ne"):
            lw = lp["w_" + lbl]
            ent["w1"].append(_pad2(lw[0]["W"], D, D).astype(jnp.bfloat16))
            ent["b1"].append(_pad1(lw[0]["b"], D))
            ent["w2"].append(_pad2(lw[1]["W"], D, D).astype(jnp.bfloat16))
            ent["b2"].append(_pad1(lw[1]["b"], D))
            ent["w3"].append(_pad2(lw[2]["W"], D, D).astype(jnp.bfloat16))
        for nm in ent:
            ent[nm] = jnp.stack(ent[nm])
        ent["hw1"] = lp["h"][0]["W"]
        ent["hb1"] = lp["h"][0]["b"][None, :]
        ent["hw2"] = lp["h"][1]["W"]
        ent["gw1"] = jnp.stack([lp["g_" + l][0]["W"] for l in
                                ("same", "anti", "ne")])
        ent["gb1"] = jnp.stack([lp["g_" + l][0]["b"][None, :] for l in
                                ("same", "anti", "ne")])
        ent["gw2"] = jnp.stack([lp["g_" + l][1]["W"] for l in
                                ("same", "anti", "ne")])
        wl.append(ent)

    posg = _make_gather(16, 6 * EP)(tpos, pos_idx)
    edata = _expand_call(posg)
    y_gath = _make_gather(D, EP, jnp.bfloat16)(
        params["Y"].astype(jnp.bfloat16), yg_idx)

    elec = jnp.broadcast_to(params["X"][0], (NE, D))
    row_gather = _make_gather(D, EP, jnp.bfloat16)
    scat = _make_scatter()

    for ent in wl:
        hx = _mlp2_call(elec, ent["hw1"], ent["hb1"], ent["hw2"],
                        jnp.bfloat16)
        zs = []
        for ti, lbl in enumerate(("same", "anti", "ne")):
            gth = y_gath if lbl == "ne" else row_gather(hx, sg_idx[lbl])
            weh = _we_call(edata, ti, gth, ent["w1"][ti], ent["b1"][ti],
                           ent["w2"][ti], ent["b2"][ti], ent["w3"][ti])
            zs.append(scat(weh, scat_idx[lbl]))
        elec = _upd_call(elec, zs, ent["gw1"], ent["gb1"], ent["gw2"])
    return elec


# speedup vs baseline: 1.1785x; 1.1785x over previous
import functools

import numpy as np
import jax
import jax.numpy as jnp
from jax import lax
from jax.experimental import pallas as pl
from jax.experimental.pallas import tpu as pltpu
from jax.experimental.pallas import tpu_sc as plsc

NE = 10000
NN = 1000
EV = 160000
EP = 163840
D = 128
ROWB = 1024
NB_T = EP // ROWB
ZR = 10240
DUMMY = 10200
CUTOFF = 10.0
DIST_FEAT = 16
LOG2 = float(np.log(2.0))

def _ssp(x):
    return jnp.maximum(x, 0.0) + jnp.log1p(jnp.exp(-jnp.abs(x))) - LOG2


def _pad2(w, r, c):
    return jnp.zeros((r, c), jnp.float32).at[: w.shape[0], : w.shape[1]].set(w)


def _pad1(b, c):
    return jnp.zeros((1, c), jnp.float32).at[0, : b.shape[0]].set(b)


def _expand_body(s_ref, r_ref, o_ref):
    li = lax.broadcasted_iota(jnp.int32, (1, D), 1)
    q = (li % DIST_FEAT).astype(jnp.float32) / (DIST_FEAT - 1.0)
    valid = li < 7 * DIST_FEAT
    mus = jnp.where(valid, CUTOFF * q * q, 0.0)
    sig = (1.0 + CUTOFF * q) / 7.0
    isig = jnp.where(valid, 1.0 / (sig * sig), 0.0)
    cols = []
    d_sq = None
    for i in range(3):
        di = r_ref[:, i : i + 1] - s_ref[:, i : i + 1]
        d_sq = di * di if d_sq is None else d_sq + di * di
        cols.append(jnp.maximum(di, 0.0))
        cols.append(jnp.maximum(-di, 0.0))
    cols.append(jnp.sqrt(d_sq))
    x = jnp.concatenate(
        [jnp.broadcast_to(c, (ROWB, DIST_FEAT)) for c in cols]
        + [jnp.zeros((ROWB, DIST_FEAT), jnp.float32)],
        axis=1,
    )
    env = x * x * jnp.exp(-x)
    dm = x - mus
    o_ref[...] = (env * jnp.exp(-(dm * dm) * isig)).astype(jnp.bfloat16)


def _expand_call(posg):
    return pl.pallas_call(
        _expand_body,
        grid=(3 * NB_T,),
        in_specs=[
            pl.BlockSpec((ROWB, 16), lambda i: (3 * NB_T + i, 0)),
            pl.BlockSpec((ROWB, 16), lambda i: (i, 0)),
        ],
        out_specs=pl.BlockSpec((ROWB, D), lambda i: (i, 0)),
        out_shape=jax.ShapeDtypeStruct((3 * EP, D), jnp.bfloat16),
    )(posg, posg)


def _mlp2_body(x_ref, w1_ref, b1_ref, w2_ref, o_ref):
    a = _ssp(jnp.dot(x_ref[...], w1_ref[...],
                     preferred_element_type=jnp.float32) + b1_ref[...])
    o_ref[...] = jnp.dot(a, w2_ref[...],
                         preferred_element_type=jnp.float32).astype(o_ref.dtype)


def _mlp2_call(x, w1, b1, w2, out_dtype=jnp.float32):
    n = x.shape[0]
    blk = 1000
    return pl.pallas_call(
        _mlp2_body,
        grid=(n // blk,),
        in_specs=[
            pl.BlockSpec((blk, D), lambda i: (i, 0)),
            pl.BlockSpec((D, D), lambda i: (0, 0)),
            pl.BlockSpec((1, D), lambda i: (0, 0)),
            pl.BlockSpec((D, D), lambda i: (0, 0)),
        ],
        out_specs=pl.BlockSpec((blk, D), lambda i: (i, 0)),
        out_shape=jax.ShapeDtypeStruct((n, D), out_dtype),
    )(x, w1, b1, w2)


def _we_body(e_ref, g_ref, w1, b1, w2, b2, w3, o_ref):
    a = _ssp(jnp.dot(e_ref[...], w1[...], preferred_element_type=jnp.float32)
             + b1[...]).astype(jnp.bfloat16)
    a = _ssp(jnp.dot(a, w2[...], preferred_element_type=jnp.float32)
             + b2[...]).astype(jnp.bfloat16)
    o_ref[...] = (jnp.dot(a, w3[...], preferred_element_type=jnp.float32)
                  * g_ref[...].astype(jnp.float32))


def _we_call(edata, t_idx, gath, w1, b1, w2, b2, w3):
    return pl.pallas_call(
        _we_body,
        grid=(NB_T,),
        in_specs=[
            pl.BlockSpec((ROWB, D), lambda i: (t_idx * NB_T + i, 0)),
            pl.BlockSpec((ROWB, D), lambda i: (i, 0)),
            pl.BlockSpec((D, D), lambda i: (0, 0)),
            pl.BlockSpec((1, D), lambda i: (0, 0)),
            pl.BlockSpec((D, D), lambda i: (0, 0)),
            pl.BlockSpec((1, D), lambda i: (0, 0)),
            pl.BlockSpec((D, D), lambda i: (0, 0)),
        ],
        out_specs=pl.BlockSpec((ROWB, D), lambda i: (i, 0)),
        out_shape=jax.ShapeDtypeStruct((EP, D), jnp.float32),
    )(edata, gath, w1, b1, w2, b2, w3)


def _upd_body(elec_ref, z0, z1, z2, w1, b1, w2, o_ref):
    acc = elec_ref[...]
    for t, zr in enumerate((z0, z1, z2)):
        zt = zr[0] + zr[1]
        a = _ssp(jnp.dot(zt, w1[t], preferred_element_type=jnp.float32) + b1[t])
        acc = acc + jnp.dot(a, w2[t], preferred_element_type=jnp.float32)
    o_ref[...] = acc


def _upd_call(elec, zs, w1, b1, w2):
    blk = 1000
    zspec = pl.BlockSpec((2, blk, D), lambda i: (0, i, 0))
    return pl.pallas_call(
        _upd_body,
        grid=(NE // blk,),
        in_specs=[
            pl.BlockSpec((blk, D), lambda i: (i, 0)),
            zspec, zspec, zspec,
            pl.BlockSpec((3, D, D), lambda i: (0, 0, 0)),
            pl.BlockSpec((3, 1, D), lambda i: (0, 0, 0)),
            pl.BlockSpec((3, D, D), lambda i: (0, 0, 0)),
        ],
        out_specs=pl.BlockSpec((blk, D), lambda i: (i, 0)),
        out_shape=jax.ShapeDtypeStruct((NE, D), jnp.float32),
    )(elec, zs[0], zs[1], zs[2], w1, b1, w2)


def _mesh():
    return plsc.VectorSubcoreMesh(core_axis_name="c", subcore_axis_name="s")


NBUF = 4


@functools.partial(functools.lru_cache)
def _make_gather(dcols, b_total, dtype=jnp.float32):
    per_w = b_total // 32
    n_chunks = per_w // 128
    n_groups = n_chunks // NBUF

    @functools.partial(
        pl.kernel,
        out_type=jax.ShapeDtypeStruct((b_total, dcols), dtype),
        mesh=_mesh(),
        compiler_params=pltpu.CompilerParams(use_tc_tiling_on_sc=False),
        scratch_types=[
            pltpu.VMEM((n_chunks, 128), jnp.int32),
            pltpu.VMEM((NBUF, 128, dcols), dtype),
        ] + [pltpu.SemaphoreType.DMA] * NBUF,
    )
    def gk(table_hbm, idx_hbm, out_hbm, idx_all, rows, *sems):
        c = lax.axis_index("c")
        s = lax.axis_index("s")
        wid = s * 2 + c
        base = wid * per_w
        pltpu.sync_copy(idx_hbm.at[pl.ds(wid * n_chunks, n_chunks)], idx_all)

        def gstart(b, g):
            pltpu.async_copy(table_hbm.at[idx_all.at[g]], rows.at[b], sems[b])

        def gwait(b):
            pltpu.make_async_copy(
                out_hbm.at[pl.ds(base, 128)], rows.at[b], sems[b]).wait()

        for b in range(NBUF):
            gstart(b, b)

        def grp(k, carry):
            for b in range(NBUF):
                gwait(b)
                g = k * NBUF + b
                pltpu.async_copy(
                    rows.at[b], out_hbm.at[pl.ds(base + g * 128, 128)], sems[b])
            for b in range(NBUF):
                gwait(b)

            @pl.when(k + 1 < n_groups)
            def _():
                for b in range(NBUF):
                    gstart(b, (k + 1) * NBUF + b)
            return carry

        lax.fori_loop(0, n_groups, grp, 0)

    return gk


@functools.partial(functools.lru_cache)
def _make_scatter():
    per_w = EP // 32
    n_chunks = per_w // 128
    snbuf = 2
    n_groups = n_chunks // snbuf
    tile_rows = ZR // 16

    @functools.partial(
        pl.kernel,
        out_type=jax.ShapeDtypeStruct((2, ZR, D), jnp.float32),
        mesh=_mesh(),
        scratch_types=[
            pltpu.VMEM((n_chunks, 128), jnp.int32),
            pltpu.VMEM((snbuf, 128, D), jnp.float32),
            pltpu.VMEM_SHARED((ZR, D), jnp.float32),
        ] + [pltpu.SemaphoreType.DMA] * snbuf,
    )
    def sk(weh, idx_hbm, out_hbm, idx_all, rows, acc, *sems):
        c = lax.axis_index("c")
        s = lax.axis_index("s")
        wid = c * 16 + s
        base = c * (EP // 2) + s * per_w
        pltpu.sync_copy(idx_hbm.at[pl.ds(wid * n_chunks, n_chunks)], idx_all)

        def zb(i, carry):
            for k in range(8):
                rows[0, i, pl.ds(k * 16, 16)] = jnp.zeros((16,), jnp.float32)
            return carry

        lax.fori_loop(0, 128, zb, 0)
        tile_r0 = s * tile_rows
        for bb in range(tile_rows // 128):
            pltpu.sync_copy(rows.at[0], acc.at[pl.ds(tile_r0 + bb * 128, 128)])
        plsc.subcore_barrier()

        def lstart(b, g):
            pltpu.async_copy(
                weh.at[pl.ds(base + g * 128, 128)], rows.at[b], sems[b])

        def swait(b):
            pltpu.make_async_copy(
                weh.at[pl.ds(base, 128)], rows.at[b], sems[b]).wait()

        for b in range(snbuf):
            lstart(b, b)

        def grp(k, carry):
            for b in range(snbuf):
                swait(b)
                g = k * snbuf + b
                pltpu.async_copy(rows.at[b], acc.at[idx_all.at[g]], sems[b],
                                 add=True)
            for b in range(snbuf):
                swait(b)

            @pl.when(k + 1 < n_groups)
            def _():
                for b in range(snbuf):
                    lstart(b, (k + 1) * snbuf + b)
            return carry

        lax.fori_loop(0, n_groups, grp, 0)
        plsc.subcore_barrier()

        for bb in range(tile_rows // 128):
            r0 = tile_r0 + bb * 128
            pltpu.sync_copy(acc.at[pl.ds(r0, 128)], rows.at[0])
            pltpu.sync_copy(rows.at[0], out_hbm.at[c, pl.ds(r0, 128)])

    return sk


def _pad_idx(a, n, fill):
    a = a.astype(jnp.int32)
    return jnp.concatenate([a, jnp.full((n - a.shape[0],), fill, jnp.int32)])


def _chunked(idx):
    return idx.reshape(idx.shape[0] // 128, 128)


def kernel(params, rs, coords, senders_ne, receivers_ne, senders_same,
           receivers_same, senders_anti, receivers_anti):
    senders = {"same": senders_same, "anti": senders_anti, "ne": senders_ne}
    receivers = {"same": receivers_same, "anti": receivers_anti,
                 "ne": receivers_ne}

    tpos = jnp.concatenate(
        [_pad2(rs, NE, 16), _pad2(coords, NN, 16)], axis=0)
    rpos_idx = jnp.concatenate(
        [_pad_idx(receivers[l], EP, receivers[l][0]) for l in
         ("same", "anti", "ne")])
    spos_idx = jnp.concatenate([
        _pad_idx(senders["same"], EP, senders["same"][0]),
        _pad_idx(senders["anti"], EP, senders["anti"][0]),
        _pad_idx(senders["ne"], EP, senders["ne"][0]) + NE,
    ])
    pos_idx = _chunked(jnp.concatenate([rpos_idx, spos_idx]))
    sg_idx = {l: _chunked(_pad_idx(senders[l], EP, 0)) for l in ("same", "anti")}
    yg_idx = _chunked(_pad_idx(senders["ne"], EP, 0))
    scat_idx = {l: _chunked(_pad_idx(receivers[l], EP, DUMMY))
                for l in ("same", "anti", "ne")}

    wl = []
    for lp in params["layers"]:
        ent = {}
        for nm in ("w1", "b1", "w2", "b2", "w3"):
            ent[nm] = []
        for lbl in ("same", "anti", "ne"):
            lw = lp["w_" + lbl]
            ent["w1"].append(_pad2(lw[0]["W"], D, D).astype(jnp.bfloat16))
            ent["b1"].append(_pad1(lw[0]["b"], D))
            ent["w2"].append(_pad2(lw[1]["W"], D, D).astype(jnp.bfloat16))
            ent["b2"].append(_pad1(lw[1]["b"], D))
            ent["w3"].append(_pad2(lw[2]["W"], D, D).astype(jnp.bfloat16))
        for nm in ent:
            ent[nm] = jnp.stack(ent[nm])
        ent["hw1"] = lp["h"][0]["W"]
        ent["hb1"] = lp["h"][0]["b"][None, :]
        ent["hw2"] = lp["h"][1]["W"]
        ent["gw1"] = jnp.stack([lp["g_" + l][0]["W"] for l in
                                ("same", "anti", "ne")])
        ent["gb1"] = jnp.stack([lp["g_" + l][0]["b"][None, :] for l in
                                ("same", "anti", "ne")])
        ent["gw2"] = jnp.stack([lp["g_" + l][1]["W"] for l in
                                ("same", "anti", "ne")])
        wl.append(ent)

    posg = _make_gather(16, 6 * EP)(tpos, pos_idx)
    edata = _expand_call(posg)
    y_gath = _make_gather(D, EP)(params["Y"], yg_idx)

    elec = jnp.broadcast_to(params["X"][0], (NE, D))
    row_gather = _make_gather(D, EP)
    scat = _make_scatter()

    for ent in wl:
        hx = _mlp2_call(elec, ent["hw1"], ent["hb1"], ent["hw2"])
        zs = []
        for ti, lbl in enumerate(("same", "anti", "ne")):
            gth = y_gath if lbl == "ne" else row_gather(hx, sg_idx[lbl])
            weh = _we_call(edata, ti, gth, ent["w1"][ti], ent["b1"][ti],
                           ent["w2"][ti], ent["b2"][ti], ent["w3"][ti])
            zs.append(scat(weh, scat_idx[lbl]))
        elec = _upd_call(elec, zs, ent["gw1"], ent["gb1"], ent["gw2"])
    return elec
